# baseline (speedup 1.0000x reference)
import functools

import jax
import jax.numpy as jnp
from jax import lax
from jax.experimental import pallas as pl
from jax.experimental.pallas import tpu as pltpu
from jax.experimental.pallas import tpu_sc as plsc

_B, _F, _V, _D = 16384, 26, 100000, 32
_EPS = 1e-5
_R = _B * _F
_C = 13 + _F * _D

_NW = 32
_RW = _R // _NW
_G = 128
_GW = _RW // _G
_GC = 13
_NCH = _GW // _GC
_CH = _GC * _G

_BR = 1024
_NB = _B // _BR


def _sc_gather(table2d, idx2d):
    mesh = plsc.VectorSubcoreMesh(core_axis_name="c", subcore_axis_name="s")

    @functools.partial(
        pl.kernel,
        mesh=mesh,
        compiler_params=pltpu.CompilerParams(use_tc_tiling_on_sc=False),
        out_type=jax.ShapeDtypeStruct((_R, _D), jnp.float32),
        scratch_types=[
            pltpu.VMEM((_GW, _G), jnp.int32),
            pltpu.VMEM((_CH, _D), jnp.float32),
            pltpu.SemaphoreType.DMA,
        ],
    )
    def k(table_hbm, idx_hbm, out_hbm, idx_v, rows_v, sem):
        wid = lax.axis_index("s") * 2 + lax.axis_index("c")
        pltpu.sync_copy(idx_hbm.at[pl.ds(wid * _GW, _GW)], idx_v)
        row_base = wid * _RW

        def chunk(c, carry):
            cps = [
                pltpu.async_copy(
                    table_hbm.at[idx_v.at[c * _GC + j]],
                    rows_v.at[pl.ds(j * _G, _G)],
                    sem,
                )
                for j in range(_GC)
            ]
            for cp in cps:
                cp.wait()
            pltpu.sync_copy(rows_v, out_hbm.at[pl.ds(row_base + c * _CH, _CH)])
            return carry

        lax.fori_loop(0, _NCH, chunk, 0)

    return k(table2d, idx2d)


def _stats_body(num_ref, emb_ref, on_ref, oe_ref):
    @pl.when(pl.program_id(0) == 0)
    def _init():
        on_ref[...] = jnp.zeros_like(on_ref)
        oe_ref[...] = jnp.zeros_like(oe_ref)

    xn = num_ref[...]
    xe = emb_ref[...]
    on_ref[...] += jnp.stack([jnp.sum(xn, 0), jnp.sum(xn * xn, 0)])
    oe_ref[...] += jnp.stack([jnp.sum(xe, 0), jnp.sum(xe * xe, 0)])


def _tc_stats(numeric, emb3):
    return pl.pallas_call(
        _stats_body,
        grid=(_NB,),
        in_specs=[
            pl.BlockSpec((_BR, 13), lambda i: (i, 0)),
            pl.BlockSpec((_BR, _F, _D), lambda i: (i, 0, 0)),
        ],
        out_specs=[
            pl.BlockSpec((2, 13), lambda i: (0, 0)),
            pl.BlockSpec((2, _F, _D), lambda i: (0, 0, 0)),
        ],
        out_shape=[
            jax.ShapeDtypeStruct((2, 13), jnp.float32),
            jax.ShapeDtypeStruct((2, _F, _D), jnp.float32),
        ],
    )(numeric, emb3)


def _norm_body(num_ref, emb_ref, sn_ref, se_ref, gn_ref, ge_ref, bn_ref, be_ref, o_ref):
    inv_b = 1.0 / _B
    sn = sn_ref[...]
    mean_n = sn[0:1] * inv_b
    var_n = sn[1:2] * inv_b - mean_n * mean_n
    scale_n = gn_ref[...] * lax.rsqrt(var_n + _EPS)
    shift_n = bn_ref[...] - mean_n * scale_n

    se = se_ref[...]
    mean_e = se[0:1] * inv_b
    var_e = se[1:2] * inv_b - mean_e * mean_e
    scale_e = ge_ref[...] * lax.rsqrt(var_e + _EPS)
    shift_e = be_ref[...] - mean_e * scale_e

    yn = num_ref[...] * scale_n + shift_n
    ye = emb_ref[...] * scale_e + shift_e
    o_ref[...] = jnp.concatenate([yn, ye.reshape(_BR, _F * _D)], axis=1)


def _tc_norm(numeric, emb3, stats_n, stats_e, gn, ge, bn, be):
    return pl.pallas_call(
        _norm_body,
        grid=(_NB,),
        in_specs=[
            pl.BlockSpec((_BR, 13), lambda i: (i, 0)),
            pl.BlockSpec((_BR, _F, _D), lambda i: (i, 0, 0)),
            pl.BlockSpec((2, 13), lambda i: (0, 0)),
            pl.BlockSpec((2, _F, _D), lambda i: (0, 0, 0)),
            pl.BlockSpec((1, 13), lambda i: (0, 0)),
            pl.BlockSpec((1, _F, _D), lambda i: (0, 0, 0)),
            pl.BlockSpec((1, 13), lambda i: (0, 0)),
            pl.BlockSpec((1, _F, _D), lambda i: (0, 0, 0)),
        ],
        out_specs=pl.BlockSpec((_BR, _C), lambda i: (i, 0)),
        out_shape=jax.ShapeDtypeStruct((_B, _C), jnp.float32),
    )(numeric, emb3, stats_n, stats_e, gn, ge, bn, be)


def kernel(numeric, indices, tables, gamma, beta):
    table2d = tables.reshape(_F * _V, _D)
    flat = (
        indices.astype(jnp.int32) + (jnp.arange(_F, dtype=jnp.int32) * _V)[None, :]
    ).reshape(_R // _G, _G)
    rows = _sc_gather(table2d, flat)
    emb3 = rows.reshape(_B, _F, _D)
    stats_n, stats_e = _tc_stats(numeric, emb3)
    gn = gamma[:13].reshape(1, 13)
    ge = gamma[13:].reshape(1, _F, _D)
    bn = beta[:13].reshape(1, 13)
    be = beta[13:].reshape(1, _F, _D)
    return _tc_norm(numeric, emb3, stats_n, stats_e, gn, ge, bn, be)


# speedup vs baseline: 1.0828x; 1.0828x over previous
import functools

import jax
import jax.numpy as jnp
from jax import lax
from jax.experimental import pallas as pl
from jax.experimental.pallas import tpu as pltpu
from jax.experimental.pallas import tpu_sc as plsc

_B, _F, _V, _D = 16384, 26, 100000, 32
_EPS = 1e-5
_R = _B * _F
_C = 13 + _F * _D
_RF = _R * _D // 128

_NW = 32
_RW = _R // _NW
_G = 128
_GW = _RW // _G
_GC = 13
_NCH = _GW // _GC
_CH = _GC * _G
_CHF = _CH * _D // 128

_BR = 512
_NB = _B // _BR
_BRF = _BR * _F * _D // 128


def _sc_gather(table2d, idx2d):
    mesh = plsc.VectorSubcoreMesh(core_axis_name="c", subcore_axis_name="s")

    @functools.partial(
        pl.kernel,
        mesh=mesh,
        compiler_params=pltpu.CompilerParams(use_tc_tiling_on_sc=False),
        out_type=jax.ShapeDtypeStruct((_R, _D), jnp.float32),
        scratch_types=[
            pltpu.VMEM((_GW, _G), jnp.int32),
            pltpu.VMEM((_CH, _D), jnp.float32),
            pltpu.SemaphoreType.DMA,
        ],
    )
    def k(table_hbm, idx_hbm, out_hbm, idx_v, rows_v, sem):
        wid = lax.axis_index("s") * 2 + lax.axis_index("c")
        pltpu.sync_copy(idx_hbm.at[pl.ds(wid * _GW, _GW)], idx_v)
        row_base = wid * _RW
        out_rows = out_hbm

        def chunk(c, carry):
            cps = [
                pltpu.async_copy(
                    table_hbm.at[idx_v.at[c * _GC + j]],
                    rows_v.at[pl.ds(j * _G, _G)],
                    sem,
                )
                for j in range(_GC)
            ]
            for cp in cps:
                cp.wait()
            pltpu.sync_copy(rows_v, out_rows.at[pl.ds(row_base + c * _CH, _CH)])
            return carry

        lax.fori_loop(0, _NCH, chunk, 0)

    return k(table2d, idx2d)


def _stats_body(num_ref, emb_ref, on_ref, oe_ref):
    @pl.when(pl.program_id(0) == 0)
    def _init():
        on_ref[...] = jnp.zeros_like(on_ref)
        oe_ref[...] = jnp.zeros_like(oe_ref)

    xn = num_ref[...]
    xe = emb_ref[...].reshape(_BRF // 13, 13, 128)
    on_ref[...] += jnp.stack([jnp.sum(xn, 0), jnp.sum(xn * xn, 0)])
    oe_ref[...] += jnp.stack([jnp.sum(xe, 0), jnp.sum(xe * xe, 0)])


def _tc_stats(numeric, embf):
    return pl.pallas_call(
        _stats_body,
        grid=(_NB,),
        in_specs=[
            pl.BlockSpec((_BR, 13), lambda i: (i, 0)),
            pl.BlockSpec((_BRF, 128), lambda i: (i, 0)),
        ],
        out_specs=[
            pl.BlockSpec((2, 13), lambda i: (0, 0)),
            pl.BlockSpec((2, 13, 128), lambda i: (0, 0, 0)),
        ],
        out_shape=[
            jax.ShapeDtypeStruct((2, 13), jnp.float32),
            jax.ShapeDtypeStruct((2, 13, 128), jnp.float32),
        ],
    )(numeric, embf)


def _fold832(p13):
    a = jnp.roll(p13, -6, axis=0)
    b = jnp.roll(p13, -7, axis=0)
    other = jnp.concatenate([a[:, 64:], b[:, :64]], axis=1)
    return p13 + other


def _norm_body(num_ref, emb_ref, sn_ref, se_ref, gn_ref, ge_ref, bn_ref, be_ref, o_ref):
    inv_b = 1.0 / _B
    sn = sn_ref[...]
    mean_n = sn[0:1] * inv_b
    var_n = sn[1:2] * inv_b - mean_n * mean_n
    scale_n = gn_ref[...] * lax.rsqrt(var_n + _EPS)
    shift_n = bn_ref[...] - mean_n * scale_n

    s_e = _fold832(se_ref[0])
    q_e = _fold832(se_ref[1])
    mean_e = s_e * inv_b
    var_e = q_e * inv_b - mean_e * mean_e
    scale_e = ge_ref[...] * lax.rsqrt(var_e + _EPS)
    shift_e = be_ref[...] - mean_e * scale_e

    yn = num_ref[...] * scale_n + shift_n
    xe = emb_ref[...].reshape(_BRF // 13, 13, 128)
    y3 = xe * scale_e[None] + shift_e[None]
    pieces = [yn]
    for j in range(6):
        ev = y3[:, j, :]
        od = jnp.concatenate([y3[:, 6 + j, 64:], y3[:, 7 + j, :64]], axis=1)
        pieces.append(jnp.stack([ev, od], axis=1).reshape(_BR, 128))
    ev6 = y3[:, 6, :64]
    od6 = y3[:, 12, 64:]
    pieces.append(jnp.stack([ev6, od6], axis=1).reshape(_BR, 64))
    o_ref[...] = jnp.concatenate(pieces, axis=1)


def _tc_norm(numeric, embf, stats_n, stats_e, gn, ge, bn, be):
    return pl.pallas_call(
        _norm_body,
        grid=(_NB,),
        in_specs=[
            pl.BlockSpec((_BR, 13), lambda i: (i, 0)),
            pl.BlockSpec((_BRF, 128), lambda i: (i, 0)),
            pl.BlockSpec((2, 13), lambda i: (0, 0)),
            pl.BlockSpec((2, 13, 128), lambda i: (0, 0, 0)),
            pl.BlockSpec((1, 13), lambda i: (0, 0)),
            pl.BlockSpec((13, 128), lambda i: (0, 0)),
            pl.BlockSpec((1, 13), lambda i: (0, 0)),
            pl.BlockSpec((13, 128), lambda i: (0, 0)),
        ],
        out_specs=pl.BlockSpec((_BR, _C), lambda i: (i, 0)),
        out_shape=jax.ShapeDtypeStruct((_B, _C), jnp.float32),
    )(numeric, embf, stats_n, stats_e, gn, ge, bn, be)


def kernel(numeric, indices, tables, gamma, beta):
    table2d = tables.reshape(_F * _V, _D)
    flat = (
        indices.astype(jnp.int32) + (jnp.arange(_F, dtype=jnp.int32) * _V)[None, :]
    ).reshape(_R // _G, _G)
    embf = _sc_gather(table2d, flat).reshape(_RF, 128)
    gn = gamma[:13].reshape(1, 13)
    bn = beta[:13].reshape(1, 13)
    ge = jnp.tile(gamma[13:], 2).reshape(13, 128)
    be = jnp.tile(beta[13:], 2).reshape(13, 128)
    stats_n, stats_e = _tc_stats(numeric, embf)
    return _tc_norm(numeric, embf, stats_n, stats_e, gn, ge, bn, be)


# speedup vs baseline: 2.7256x; 2.5173x over previous
import functools

import jax
import jax.numpy as jnp
from jax import lax
from jax.experimental import pallas as pl
from jax.experimental.pallas import tpu as pltpu
from jax.experimental.pallas import tpu_sc as plsc

_B, _F, _V, _D = 16384, 26, 100000, 32
_EPS = 1e-5
_C = 13 + _F * _D
_NW = 32

_BL = 2048
_NB = _B // _BL


def _sc_gather(tablesT, idxT3):
    mesh = plsc.VectorSubcoreMesh(core_axis_name="c", subcore_axis_name="s")

    @functools.partial(
        pl.kernel,
        mesh=mesh,
        compiler_params=pltpu.CompilerParams(needs_layout_passes=False),
        out_type=jax.ShapeDtypeStruct((_F * _D, _B), jnp.float32),
        scratch_types=[
            pltpu.VMEM((_V,), jnp.float32),
            pltpu.VMEM((16, 128), jnp.int32),
            pltpu.VMEM((_B,), jnp.float32),
            pltpu.SemaphoreType.DMA,
        ],
    )
    def k(tab_hbm, idx_hbm, out_hbm, plane_v, idx_v, col_v, sem):
        wid = lax.axis_index("s") * 2 + lax.axis_index("c")
        for t in range(_F):
            c = t * _NW + wid
            pltpu.sync_copy(tab_hbm.at[c], plane_v)

            def hblock(h, carry):
                pltpu.sync_copy(idx_hbm.at[t, pl.ds(h * 16, 16)], idx_v)

                def irow(i, carry2):
                    base = h * 2048 + i * 128
                    for s in range(8):
                        v16 = idx_v[i, pl.ds(s * 16, 16)]
                        x = plsc.load_gather(plane_v, [v16])
                        col_v[pl.ds(base + s * 16, 16)] = x
                    return carry2

                lax.fori_loop(0, 16, irow, 0)
                return carry

            lax.fori_loop(0, 8, hblock, 0)
            pltpu.sync_copy(col_v, out_hbm.at[c])

    return k(tablesT, idxT3)


def _stats_body(num_ref, emb_ref, o_ref):
    @pl.when(pl.program_id(0) == 0)
    def _init():
        o_ref[...] = jnp.zeros_like(o_ref)

    xn = num_ref[...]
    xe = emb_ref[...]
    s = jnp.concatenate(
        [jnp.sum(xn, 1, keepdims=True), jnp.sum(xe, 1, keepdims=True)], axis=0
    )
    q = jnp.concatenate(
        [jnp.sum(xn * xn, 1, keepdims=True), jnp.sum(xe * xe, 1, keepdims=True)],
        axis=0,
    )
    o_ref[...] += jnp.concatenate([s, q], axis=1)


def _tc_stats(numericT, embT):
    return pl.pallas_call(
        _stats_body,
        grid=(_NB,),
        in_specs=[
            pl.BlockSpec((13, _BL), lambda i: (0, i)),
            pl.BlockSpec((_F * _D, _BL), lambda i: (0, i)),
        ],
        out_specs=pl.BlockSpec((_C, 2), lambda i: (0, 0)),
        out_shape=jax.ShapeDtypeStruct((_C, 2), jnp.float32),
    )(numericT, embT)


def _norm_body(num_ref, emb_ref, st_ref, g_ref, b_ref, o_ref):
    inv_b = 1.0 / _B
    st = st_ref[...]
    mean = st[:, 0:1] * inv_b
    var = st[:, 1:2] * inv_b - mean * mean
    scale = g_ref[...] * lax.rsqrt(var + _EPS)
    shift = b_ref[...] - mean * scale
    x = jnp.concatenate([num_ref[...], emb_ref[...]], axis=0)
    o_ref[...] = x * scale + shift


def _tc_norm(numericT, embT, stats, g2, b2):
    return pl.pallas_call(
        _norm_body,
        grid=(_NB,),
        in_specs=[
            pl.BlockSpec((13, _BL), lambda i: (0, i)),
            pl.BlockSpec((_F * _D, _BL), lambda i: (0, i)),
            pl.BlockSpec((_C, 2), lambda i: (0, 0)),
            pl.BlockSpec((_C, 1), lambda i: (0, 0)),
            pl.BlockSpec((_C, 1), lambda i: (0, 0)),
        ],
        out_specs=pl.BlockSpec((_C, _BL), lambda i: (0, i)),
        out_shape=jax.ShapeDtypeStruct((_C, _B), jnp.float32),
    )(numericT, embT, stats, g2, b2)


def kernel(numeric, indices, tables, gamma, beta):
    tablesT = tables.transpose(0, 2, 1).reshape(_F * _D, _V)
    idxT3 = indices.T.reshape(_F, 128, 128)
    embT = _sc_gather(tablesT, idxT3)
    numericT = numeric.T
    stats = _tc_stats(numericT, embT)
    outT = _tc_norm(numericT, embT, stats, gamma.reshape(_C, 1), beta.reshape(_C, 1))
    return outT.T


# speedup vs baseline: 2.7573x; 1.0116x over previous
import functools

import jax
import jax.numpy as jnp
from jax import lax
from jax.experimental import pallas as pl
from jax.experimental.pallas import tpu as pltpu
from jax.experimental.pallas import tpu_sc as plsc

_B, _F, _V, _D = 16384, 26, 100000, 32
_EPS = 1e-5
_C = 13 + _F * _D
_NW = 32

_BL = 2048
_NB = _B // _BL


def _sc_gather(tablesT, idxT3, numericT):
    mesh = plsc.VectorSubcoreMesh(core_axis_name="c", subcore_axis_name="s")

    @functools.partial(
        pl.kernel,
        mesh=mesh,
        compiler_params=pltpu.CompilerParams(needs_layout_passes=False),
        out_type=(
            jax.ShapeDtypeStruct((_F * _D, _B), jnp.float32),
            jax.ShapeDtypeStruct((_C, 16), jnp.float32),
        ),
        scratch_types=[
            pltpu.VMEM((_V,), jnp.float32),
            pltpu.VMEM((16, 128), jnp.int32),
            pltpu.VMEM((_B,), jnp.float32),
            pltpu.VMEM((16,), jnp.float32),
            pltpu.SemaphoreType.DMA,
        ],
    )
    def k(tab_hbm, idx_hbm, num_hbm, out_hbm, st_hbm, plane_v, idx_v, col_v,
          stage_v, sem):
        wid = lax.axis_index("s") * 2 + lax.axis_index("c")
        lanes = lax.iota(jnp.int32, 16)
        zero16 = lax.broadcast(jnp.float32(0.0), (16,))

        def put_stats(row, acc_s, acc_q):
            rs = jnp.sum(acc_s)
            rq = jnp.sum(acc_q)
            stage_v[...] = jnp.where(lanes == 0, rs, 0.0) + jnp.where(
                lanes == 1, rq, 0.0
            )
            pltpu.sync_copy(stage_v, st_hbm.at[row])

        for t in range(_F):
            c = t * _NW + wid
            pltpu.sync_copy(tab_hbm.at[c], plane_v)

            def hblock(h, acc):
                pltpu.sync_copy(idx_hbm.at[t, pl.ds(h * 16, 16)], idx_v)

                def irow(i, acc2):
                    a_s, a_q = acc2
                    base = h * 2048 + i * 128
                    for s in range(8):
                        v16 = idx_v[i, pl.ds(s * 16, 16)]
                        x = plsc.load_gather(plane_v, [v16])
                        col_v[pl.ds(base + s * 16, 16)] = x
                        a_s = a_s + x
                        a_q = a_q + x * x
                    return (a_s, a_q)

                return lax.fori_loop(0, 16, irow, acc)

            acc_s, acc_q = lax.fori_loop(0, 8, hblock, (zero16, zero16))
            pltpu.sync_copy(col_v, out_hbm.at[c])
            put_stats(13 + c, acc_s, acc_q)

        @pl.when(wid < 13)
        def _numeric():
            pltpu.sync_copy(num_hbm.at[wid], col_v)

            def nsum(i, acc):
                a_s, a_q = acc
                x = col_v[pl.ds(i * 16, 16)]
                return (a_s + x, a_q + x * x)

            acc_s, acc_q = lax.fori_loop(0, _B // 16, nsum, (zero16, zero16))
            put_stats(wid, acc_s, acc_q)

    return k(tablesT, idxT3, numericT)


def _norm_body(num_ref, emb_ref, st_ref, g_ref, b_ref, o_ref):
    inv_b = 1.0 / _B
    st = st_ref[...]
    mean = st[:, 0:1] * inv_b
    var = st[:, 1:2] * inv_b - mean * mean
    scale = g_ref[...] * lax.rsqrt(var + _EPS)
    shift = b_ref[...] - mean * scale
    x = jnp.concatenate([num_ref[...], emb_ref[...]], axis=0)
    o_ref[...] = x * scale + shift


def _tc_norm(numericT, embT, stats, g2, b2):
    return pl.pallas_call(
        _norm_body,
        grid=(_NB,),
        in_specs=[
            pl.BlockSpec((13, _BL), lambda i: (0, i)),
            pl.BlockSpec((_F * _D, _BL), lambda i: (0, i)),
            pl.BlockSpec((_C, 16), lambda i: (0, 0)),
            pl.BlockSpec((_C, 1), lambda i: (0, 0)),
            pl.BlockSpec((_C, 1), lambda i: (0, 0)),
        ],
        out_specs=pl.BlockSpec((_C, _BL), lambda i: (0, i)),
        out_shape=jax.ShapeDtypeStruct((_C, _B), jnp.float32),
    )(numericT, embT, stats, g2, b2)


def kernel(numeric, indices, tables, gamma, beta):
    tablesT = tables.transpose(0, 2, 1).reshape(_F * _D, _V)
    idxT3 = indices.T.reshape(_F, 128, 128)
    numericT = numeric.T
    embT, stats = _sc_gather(tablesT, idxT3, numericT)
    outT = _tc_norm(numericT, embT, stats, gamma.reshape(_C, 1), beta.reshape(_C, 1))
    return outT.T


# speedup vs baseline: 4.2883x; 1.5552x over previous
import functools

import jax
import jax.numpy as jnp
from jax import lax
from jax.experimental import pallas as pl
from jax.experimental.pallas import tpu as pltpu
from jax.experimental.pallas import tpu_sc as plsc

_B, _F, _V, _D = 16384, 26, 100000, 32
_EPS = 1e-5
_C = 13 + _F * _D
_NW = 32

_BL = 2048
_NB = _B // _BL


def _sc_gather(tablesT, idxT3, numericT):
    mesh = plsc.VectorSubcoreMesh(core_axis_name="c", subcore_axis_name="s")

    @functools.partial(
        pl.kernel,
        mesh=mesh,
        compiler_params=pltpu.CompilerParams(needs_layout_passes=False),
        out_type=(
            jax.ShapeDtypeStruct((_F * _D, _B), jnp.float32),
            jax.ShapeDtypeStruct((_C, 16), jnp.float32),
        ),
        scratch_types=[
            pltpu.VMEM((_V,), jnp.float32),
            pltpu.VMEM((64, 128), jnp.int32),
            pltpu.VMEM((_B,), jnp.float32),
            pltpu.VMEM((16,), jnp.float32),
            pltpu.SemaphoreType.DMA,
        ],
    )
    def k(tab_hbm, idx_hbm, num_hbm, out_hbm, st_hbm, plane_v, idx_v, col_v,
          stage_v, sem):
        wid = lax.axis_index("s") * 2 + lax.axis_index("c")
        lanes = lax.iota(jnp.int32, 16)
        zero16 = lax.broadcast(jnp.float32(0.0), (16,))

        def put_stats(row, acc_s, acc_q):
            rs = jnp.sum(acc_s)
            rq = jnp.sum(acc_q)
            stage_v[...] = jnp.where(lanes == 0, rs, 0.0) + jnp.where(
                lanes == 1, rq, 0.0
            )
            pltpu.sync_copy(stage_v, st_hbm.at[row])

        for t in range(_F):
            c = t * _NW + wid
            pltpu.sync_copy(tab_hbm.at[c], plane_v)

            def hblock(h, acc):
                pltpu.sync_copy(idx_hbm.at[t, pl.ds(h * 64, 64)], idx_v)

                def irow(i, acc2):
                    a_s, a_q = acc2
                    base = h * 8192 + i * 128
                    for s in range(8):
                        v16 = idx_v[i, pl.ds(s * 16, 16)]
                        x = plsc.load_gather(plane_v, [v16])
                        col_v[pl.ds(base + s * 16, 16)] = x
                        a_s = a_s + x
                        a_q = a_q + x * x
                    return (a_s, a_q)

                return lax.fori_loop(0, 64, irow, acc)

            acc_s, acc_q = lax.fori_loop(0, 2, hblock, (zero16, zero16))
            pltpu.sync_copy(col_v, out_hbm.at[c])
            put_stats(13 + c, acc_s, acc_q)

        @pl.when(wid < 13)
        def _numeric():
            pltpu.sync_copy(num_hbm.at[wid], col_v)

            def nsum(i, acc):
                a_s, a_q = acc
                x = col_v[pl.ds(i * 16, 16)]
                return (a_s + x, a_q + x * x)

            acc_s, acc_q = lax.fori_loop(0, _B // 16, nsum, (zero16, zero16))
            put_stats(wid, acc_s, acc_q)

    return k(tablesT, idxT3, numericT)


def _norm_body(num_ref, emb_ref, st_ref, g_ref, b_ref, o_ref):
    inv_b = 1.0 / _B
    st = st_ref[...]
    mean = st[:, 0:1] * inv_b
    var = st[:, 1:2] * inv_b - mean * mean
    scale = g_ref[...] * lax.rsqrt(var + _EPS)
    shift = b_ref[...] - mean * scale
    x = jnp.concatenate([num_ref[...], emb_ref[...]], axis=0)
    o_ref[...] = x * scale + shift


def _tc_norm(numericT, embT, stats, g2, b2):
    return pl.pallas_call(
        _norm_body,
        grid=(_NB,),
        in_specs=[
            pl.BlockSpec((13, _BL), lambda i: (0, i)),
            pl.BlockSpec((_F * _D, _BL), lambda i: (0, i)),
            pl.BlockSpec((_C, 16), lambda i: (0, 0)),
            pl.BlockSpec((_C, 1), lambda i: (0, 0)),
            pl.BlockSpec((_C, 1), lambda i: (0, 0)),
        ],
        out_specs=pl.BlockSpec((_C, _BL), lambda i: (0, i)),
        out_shape=jax.ShapeDtypeStruct((_C, _B), jnp.float32),
    )(numericT, embT, stats, g2, b2)


def kernel(numeric, indices, tables, gamma, beta):
    tablesT = tables.transpose(0, 2, 1).reshape(_F * _D, _V)
    idxT3 = indices.T.reshape(_F, 128, 128)
    numericT = numeric.T
    embT, stats = _sc_gather(tablesT, idxT3, numericT)
    outT = _tc_norm(numericT, embT, stats, gamma.reshape(_C, 1), beta.reshape(_C, 1))
    return outT.T


# speedup vs baseline: 4.4929x; 1.0477x over previous
import functools

import jax
import jax.numpy as jnp
from jax import lax
from jax.experimental import pallas as pl
from jax.experimental.pallas import tpu as pltpu
from jax.experimental.pallas import tpu_sc as plsc

_B, _F, _V, _D = 16384, 26, 100000, 32
_EPS = 1e-5
_C = 13 + _F * _D
_NW = 32

_BL = 2048
_NB = _B // _BL


def _sc_gather(tablesT, idxT3, numericT):
    mesh = plsc.VectorSubcoreMesh(core_axis_name="c", subcore_axis_name="s")

    @functools.partial(
        pl.kernel,
        mesh=mesh,
        compiler_params=pltpu.CompilerParams(needs_layout_passes=False),
        out_type=(
            jax.ShapeDtypeStruct((_F * _D, _B), jnp.float32),
            jax.ShapeDtypeStruct((_C, 16), jnp.float32),
        ),
        scratch_types=[
            pltpu.VMEM((_V,), jnp.float32),
            pltpu.VMEM((64, 128), jnp.int32),
            pltpu.VMEM((_B,), jnp.float32),
            pltpu.VMEM((16,), jnp.float32),
            pltpu.SemaphoreType.DMA,
            pltpu.SemaphoreType.DMA,
            pltpu.SemaphoreType.DMA,
        ],
    )
    def k(tab_hbm, idx_hbm, num_hbm, out_hbm, st_hbm, plane_v, idx_v, col_v,
          stage_v, psem, isem, csem):
        wid = lax.axis_index("s") * 2 + lax.axis_index("c")
        lanes = lax.iota(jnp.int32, 16)
        zero16 = lax.broadcast(jnp.float32(0.0), (16,))

        def put_stats(row, acc_s, acc_q):
            rs = jnp.sum(acc_s)
            rq = jnp.sum(acc_q)
            stage_v[...] = jnp.where(lanes == 0, rs, 0.0) + jnp.where(
                lanes == 1, rq, 0.0
            )
            pltpu.sync_copy(stage_v, st_hbm.at[row])

        col_cp = None
        for t in range(_F):
            c = t * _NW + wid
            plane_cp = pltpu.async_copy(tab_hbm.at[c], plane_v, psem)
            idx_cp = pltpu.async_copy(idx_hbm.at[t, pl.ds(0, 64)], idx_v, isem)
            if col_cp is not None:
                col_cp.wait()
            plane_cp.wait()
            idx_cp.wait()

            def hblock(h, acc):
                @pl.when(h > 0)
                def _load():
                    pltpu.async_copy(
                        idx_hbm.at[t, pl.ds(h * 64, 64)], idx_v, isem
                    ).wait()

                def irow(i, acc2):
                    a_s, a_q = acc2
                    base = h * 8192 + i * 128
                    for s in range(8):
                        v16 = idx_v[i, pl.ds(s * 16, 16)]
                        x = plsc.load_gather(plane_v, [v16])
                        col_v[pl.ds(base + s * 16, 16)] = x
                        a_s = a_s + x
                        a_q = a_q + x * x
                    return (a_s, a_q)

                return lax.fori_loop(0, 64, irow, acc)

            acc_s, acc_q = lax.fori_loop(0, 2, hblock, (zero16, zero16))
            col_cp = pltpu.async_copy(col_v, out_hbm.at[c], csem)
            put_stats(13 + c, acc_s, acc_q)
        col_cp.wait()

        @pl.when(wid < 13)
        def _numeric():
            pltpu.sync_copy(num_hbm.at[wid], col_v)

            def nsum(i, acc):
                a_s, a_q = acc
                x = col_v[pl.ds(i * 16, 16)]
                return (a_s + x, a_q + x * x)

            acc_s, acc_q = lax.fori_loop(0, _B // 16, nsum, (zero16, zero16))
            put_stats(wid, acc_s, acc_q)

    return k(tablesT, idxT3, numericT)


def _norm_body(num_ref, emb_ref, st_ref, g_ref, b_ref, o_ref):
    inv_b = 1.0 / _B
    st = st_ref[...]
    mean = st[:, 0:1] * inv_b
    var = st[:, 1:2] * inv_b - mean * mean
    scale = g_ref[...] * lax.rsqrt(var + _EPS)
    shift = b_ref[...] - mean * scale
    x = jnp.concatenate([num_ref[...], emb_ref[...]], axis=0)
    o_ref[...] = x * scale + shift


def _tc_norm(numericT, embT, stats, g2, b2):
    return pl.pallas_call(
        _norm_body,
        grid=(_NB,),
        in_specs=[
            pl.BlockSpec((13, _BL), lambda i: (0, i)),
            pl.BlockSpec((_F * _D, _BL), lambda i: (0, i)),
            pl.BlockSpec((_C, 16), lambda i: (0, 0)),
            pl.BlockSpec((_C, 1), lambda i: (0, 0)),
            pl.BlockSpec((_C, 1), lambda i: (0, 0)),
        ],
        out_specs=pl.BlockSpec((_C, _BL), lambda i: (0, i)),
        out_shape=jax.ShapeDtypeStruct((_C, _B), jnp.float32),
    )(numericT, embT, stats, g2, b2)


def kernel(numeric, indices, tables, gamma, beta):
    tablesT = tables.transpose(0, 2, 1).reshape(_F * _D, _V)
    idxT3 = indices.T.reshape(_F, 128, 128)
    numericT = numeric.T
    embT, stats = _sc_gather(tablesT, idxT3, numericT)
    outT = _tc_norm(numericT, embT, stats, gamma.reshape(_C, 1), beta.reshape(_C, 1))
    return outT.T
